# Initial kernel scaffold; baseline (speedup 1.0000x reference)
import functools

import jax
import jax.numpy as jnp
from jax import lax
from jax.experimental import pallas as pl
from jax.experimental.pallas import tpu as pltpu
from jax.experimental.pallas import tpu_sc as plsc

N = 16384
H = 64
HH = 32
NNZ = 2684354
NS = 16
EW = 128
KB = 8
RPT = 1312
NB = RPT // KB
NNZ_PAD = NS * RPT * EW
RT_OUT = N // NS
ZR = 512
BLK = 1024


def _theta_diff_body(p_ref, lp_ref, w2t_ref, w3t_ref, bd_ref, out_ref):
    y = (
        jnp.dot(p_ref[...], w2t_ref[...], preferred_element_type=jnp.float32)
        - jnp.dot(lp_ref[...], w3t_ref[...], preferred_element_type=jnp.float32)
        + bd_ref[...]
    )
    out_ref[0] = y[:, :HH]
    out_ref[1] = y[:, HH:]


def _theta_diff(primal, last_primal, w2t, w3t, bd):
    return pl.pallas_call(
        _theta_diff_body,
        grid=(N // BLK,),
        in_specs=[
            pl.BlockSpec((BLK, H), lambda i: (i, 0)),
            pl.BlockSpec((BLK, H), lambda i: (i, 0)),
            pl.BlockSpec((H, H), lambda i: (0, 0)),
            pl.BlockSpec((H, H), lambda i: (0, 0)),
            pl.BlockSpec((1, H), lambda i: (0, 0)),
        ],
        out_specs=pl.BlockSpec((2, BLK, HH), lambda i: (0, i, 0)),
        out_shape=jax.ShapeDtypeStruct((2, N, HH), jnp.float32),
    )(primal, last_primal, w2t, w3t, bd)


def _final_body(d_ref, rhs_ref, sp_ref, w1t_ref, b1_ref, sig_ref, out_ref):
    y = (
        jnp.dot(d_ref[...], w1t_ref[...], preferred_element_type=jnp.float32)
        + b1_ref[...]
    )
    s = jnp.concatenate([sp_ref[0], sp_ref[1]], axis=1)
    y = y + sig_ref[0] * (s - rhs_ref[...])
    out_ref[...] = jnp.where(y >= 0, y, 0.01 * y)


def _final(dual, rhs, spmm2, w1t, b1, sig):
    return pl.pallas_call(
        _final_body,
        grid=(N // BLK,),
        in_specs=[
            pl.BlockSpec((BLK, H), lambda i: (i, 0)),
            pl.BlockSpec((BLK, H), lambda i: (i, 0)),
            pl.BlockSpec((2, BLK, HH), lambda i: (0, i, 0)),
            pl.BlockSpec((H, H), lambda i: (0, 0)),
            pl.BlockSpec((1, H), lambda i: (0, 0)),
            pl.BlockSpec(memory_space=pltpu.SMEM),
        ],
        out_specs=pl.BlockSpec((BLK, H), lambda i: (i, 0)),
        out_shape=jax.ShapeDtypeStruct((N, H), jnp.float32),
    )(dual, rhs, spmm2, w1t, b1, sig)


def _sc_spmm_body(
    xs_hbm, cols_hbm, rows_hbm, vals_hbm, out_hbm,
    xs_sh, acc_sh, cols_v, rows_v, vals_v, g_v, z_v, sem,
):
    c = lax.axis_index("c")
    s = lax.axis_index("s")
    row0 = s * RT_OUT

    pltpu.sync_copy(
        xs_hbm.at[c, pl.ds(row0, RT_OUT)], xs_sh.at[pl.ds(row0, RT_OUT)]
    )

    zeros16 = jnp.zeros((16,), jnp.float32)

    def _zero_row(r, carry):
        z_v[r, pl.ds(0, 16)] = zeros16
        z_v[r, pl.ds(16, 16)] = zeros16
        return carry

    lax.fori_loop(0, ZR, _zero_row, 0)
    for kz in range(RT_OUT // ZR):
        pltpu.sync_copy(z_v, acc_sh.at[pl.ds(row0 + kz * ZR, ZR)])
    plsc.subcore_barrier()

    row_j = [jnp.full((16,), j, jnp.int32) for j in range(KB)]

    def _block(b, carry):
        base = b * KB
        pltpu.sync_copy(cols_hbm.at[s, pl.ds(base, KB)], cols_v)
        pltpu.sync_copy(rows_hbm.at[s, pl.ds(base, KB)], rows_v)
        pltpu.sync_copy(vals_hbm.at[s, pl.ds(base, KB)], vals_v)
        cps = [
            pltpu.async_copy(xs_sh.at[cols_v.at[j]], g_v.at[j], sem)
            for j in range(KB)
        ]
        for cp in cps:
            cp.wait()
        for j in range(KB):

            def _mul16(k, carry, j=j):
                base16 = k * 16
                v16 = vals_v[j, pl.ds(base16, 16)]
                del v16
                for i in range(16):
                    e = base16 + i
                    vb = plsc.load_gather(
                        vals_v, [row_j[j], jnp.full((16,), 1, jnp.int32) * e]
                    )
                    for h in range(HH // 16):
                        cur = g_v[j, e, pl.ds(h * 16, 16)]
                        g_v[j, e, pl.ds(h * 16, 16)] = cur * vb
                return carry

            lax.fori_loop(0, EW // 16, _mul16, 0)
        for j in range(KB):
            pltpu.sync_copy(g_v.at[j], acc_sh.at[rows_v.at[j]], add=True)
        return carry

    lax.fori_loop(0, NB, _block, 0)

    plsc.subcore_barrier()
    pltpu.sync_copy(
        acc_sh.at[pl.ds(row0, RT_OUT)], out_hbm.at[c, pl.ds(row0, RT_OUT)]
    )


@functools.cache
def _sc_spmm():
    return pl.kernel(
        _sc_spmm_body,
        out_type=jax.ShapeDtypeStruct((2, N, HH), jnp.float32),
        mesh=plsc.VectorSubcoreMesh(core_axis_name="c", subcore_axis_name="s"),
        scratch_types=[
            pltpu.VMEM_SHARED((N, HH), jnp.float32),
            pltpu.VMEM_SHARED((N, HH), jnp.float32),
            pltpu.VMEM((KB, EW), jnp.int32),
            pltpu.VMEM((KB, EW), jnp.int32),
            pltpu.VMEM((KB, EW), jnp.float32),
            pltpu.VMEM((KB, EW, HH), jnp.float32),
            pltpu.VMEM((ZR, HH), jnp.float32),
            pltpu.SemaphoreType.DMA,
        ],
    )


def kernel(primal, last_primal, dual, cons_indices, cons_values,
           right_hand_side, W1, b1, W2, b2, W3, b3, sigma):
    rows = cons_indices[0]
    cols = cons_indices[1]
    pad = NNZ_PAD - NNZ
    cols3 = jnp.pad(cols, (0, pad)).reshape(NS, RPT, EW)
    rows3 = jnp.pad(rows, (0, pad)).reshape(NS, RPT, EW)
    vals3 = jnp.pad(cons_values, (0, pad)).reshape(NS, RPT, EW)

    xs = _theta_diff(primal, last_primal, W2.T, W3.T, (b2 - b3).reshape(1, H))
    spmm2 = _sc_spmm()(xs, cols3, rows3, vals3)
    return _final(
        dual, right_hand_side, spmm2, W1.T, b1.reshape(1, H), sigma.reshape(1)
    )


# speedup vs baseline: 10.5372x; 10.5372x over previous
import functools

import jax
import jax.numpy as jnp
from jax import lax
from jax.experimental import pallas as pl
from jax.experimental.pallas import tpu as pltpu
from jax.experimental.pallas import tpu_sc as plsc

N = 16384
H = 64
HH = 32
NNZ = 2684354
NS = 16
EW = 128
KB = 8
RPT = 1312
NB = RPT // KB
NNZ_PAD = NS * RPT * EW
RT_OUT = N // NS
ZR = 512
BLK = 1024


def _theta_diff_body(p_ref, lp_ref, w2t_ref, w3t_ref, bd_ref, out_ref):
    y = (
        jnp.dot(p_ref[...], w2t_ref[...], preferred_element_type=jnp.float32)
        - jnp.dot(lp_ref[...], w3t_ref[...], preferred_element_type=jnp.float32)
        + bd_ref[...]
    )
    out_ref[0] = y[:, :HH]
    out_ref[1] = y[:, HH:]


def _theta_diff(primal, last_primal, w2t, w3t, bd):
    return pl.pallas_call(
        _theta_diff_body,
        grid=(N // BLK,),
        in_specs=[
            pl.BlockSpec((BLK, H), lambda i: (i, 0)),
            pl.BlockSpec((BLK, H), lambda i: (i, 0)),
            pl.BlockSpec((H, H), lambda i: (0, 0)),
            pl.BlockSpec((H, H), lambda i: (0, 0)),
            pl.BlockSpec((1, H), lambda i: (0, 0)),
        ],
        out_specs=pl.BlockSpec((2, BLK, HH), lambda i: (0, i, 0)),
        out_shape=jax.ShapeDtypeStruct((2, N, HH), jnp.float32),
    )(primal, last_primal, w2t, w3t, bd)


def _final_body(d_ref, rhs_ref, sp_ref, w1t_ref, b1_ref, sig_ref, out_ref):
    y = (
        jnp.dot(d_ref[...], w1t_ref[...], preferred_element_type=jnp.float32)
        + b1_ref[...]
    )
    s = jnp.concatenate([sp_ref[0], sp_ref[1]], axis=1)
    y = y + sig_ref[0] * (s - rhs_ref[...])
    out_ref[...] = jnp.where(y >= 0, y, 0.01 * y)


def _final(dual, rhs, spmm2, w1t, b1, sig):
    return pl.pallas_call(
        _final_body,
        grid=(N // BLK,),
        in_specs=[
            pl.BlockSpec((BLK, H), lambda i: (i, 0)),
            pl.BlockSpec((BLK, H), lambda i: (i, 0)),
            pl.BlockSpec((2, BLK, HH), lambda i: (0, i, 0)),
            pl.BlockSpec((H, H), lambda i: (0, 0)),
            pl.BlockSpec((1, H), lambda i: (0, 0)),
            pl.BlockSpec(memory_space=pltpu.SMEM),
        ],
        out_specs=pl.BlockSpec((BLK, H), lambda i: (i, 0)),
        out_shape=jax.ShapeDtypeStruct((N, H), jnp.float32),
    )(dual, rhs, spmm2, w1t, b1, sig)


def _sc_spmm_body(
    xs_hbm, cols_hbm, rows_hbm, vals_hbm, out_hbm,
    xs_sh, acc_sh, cols_v, rows_v, vals_v, g_v, z_v, sem,
):
    c = lax.axis_index("c")
    s = lax.axis_index("s")
    row0 = s * RT_OUT

    pltpu.sync_copy(
        xs_hbm.at[c, pl.ds(row0, RT_OUT)], xs_sh.at[pl.ds(row0, RT_OUT)]
    )

    zeros16 = jnp.zeros((16,), jnp.float32)

    def _zero_row(r, carry):
        z_v[r, pl.ds(0, 16)] = zeros16
        z_v[r, pl.ds(16, 16)] = zeros16
        return carry

    lax.fori_loop(0, ZR, _zero_row, 0)
    for kz in range(RT_OUT // ZR):
        pltpu.sync_copy(z_v, acc_sh.at[pl.ds(row0 + kz * ZR, ZR)])
    plsc.subcore_barrier()

    row_j = [jnp.full((16,), j, jnp.int32) for j in range(KB)]

    def _block(b, carry):
        base = b * KB
        pltpu.sync_copy(cols_hbm.at[s, pl.ds(base, KB)], cols_v)
        pltpu.sync_copy(rows_hbm.at[s, pl.ds(base, KB)], rows_v)
        pltpu.sync_copy(vals_hbm.at[s, pl.ds(base, KB)], vals_v)
        cps = [
            pltpu.async_copy(xs_sh.at[cols_v.at[j]], g_v.at[j], sem)
            for j in range(KB)
        ]
        for cp in cps:
            cp.wait()
        for j in range(KB):

            def _mul16(k, carry, j=j):
                base16 = k * 16
                for i in range(16):
                    e = base16 + i
                    vb = plsc.load_gather(
                        vals_v, [row_j[j], jnp.full((16,), e, jnp.int32)]
                    )
                    for h in range(HH // 16):
                        cur = g_v[j, e, pl.ds(h * 16, 16)]
                        g_v[j, e, pl.ds(h * 16, 16)] = cur * vb
                return carry

            lax.fori_loop(0, EW // 16, _mul16, 0)
        for j in range(KB):
            pltpu.sync_copy(g_v.at[j], acc_sh.at[rows_v.at[j]], add=True)
        return carry

    lax.fori_loop(0, NB, _block, 0)

    plsc.subcore_barrier()
    pltpu.sync_copy(
        acc_sh.at[pl.ds(row0, RT_OUT)], out_hbm.at[c, pl.ds(row0, RT_OUT)]
    )


@functools.cache
def _sc_spmm():
    return pl.kernel(
        _sc_spmm_body,
        out_type=jax.ShapeDtypeStruct((2, N, HH), jnp.float32),
        mesh=plsc.VectorSubcoreMesh(core_axis_name="c", subcore_axis_name="s"),
        compiler_params=pltpu.CompilerParams(
            needs_layout_passes=False, use_tc_tiling_on_sc=False
        ),
        scratch_types=[
            pltpu.VMEM_SHARED((N, HH), jnp.float32),
            pltpu.VMEM_SHARED((N, HH), jnp.float32),
            pltpu.VMEM((KB, EW), jnp.int32),
            pltpu.VMEM((KB, EW), jnp.int32),
            pltpu.VMEM((KB, EW), jnp.float32),
            pltpu.VMEM((KB, EW, HH), jnp.float32),
            pltpu.VMEM((ZR, HH), jnp.float32),
            pltpu.SemaphoreType.DMA,
        ],
    )


def kernel(primal, last_primal, dual, cons_indices, cons_values,
           right_hand_side, W1, b1, W2, b2, W3, b3, sigma):
    rows = cons_indices[0]
    cols = cons_indices[1]
    pad = NNZ_PAD - NNZ
    cols3 = jnp.pad(cols, (0, pad)).reshape(NS, RPT, EW)
    rows3 = jnp.pad(rows, (0, pad)).reshape(NS, RPT, EW)
    vals3 = jnp.pad(cons_values, (0, pad)).reshape(NS, RPT, EW)

    xs = _theta_diff(primal, last_primal, W2.T, W3.T, (b2 - b3).reshape(1, H))
    spmm2 = _sc_spmm()(xs, cols3, rows3, vals3)
    return _final(
        dual, right_hand_side, spmm2, W1.T, b1.reshape(1, H), sigma.reshape(1)
    )
